# baseline (speedup 1.0000x reference)
import functools

import jax
import jax.numpy as jnp
from jax import lax
from jax.experimental import pallas as pl
from jax.experimental.pallas import tpu as pltpu
from jax.experimental.pallas import tpu_sc as plsc

_K = 32


def _enc_body(x_ref, w_ref, b_ref, out_ref):
    acc = lax.dot_general(
        x_ref[...], w_ref[...],
        dimension_numbers=(((1,), (1,)), ((), ())),
        preferred_element_type=jnp.float32,
    )
    out_ref[...] = acc + b_ref[...]


def _dec_body(hs_ref, w_ref, b_ref, out_ref):
    k = pl.program_id(1)

    @pl.when(k == 0)
    def _():
        out_ref[...] = jnp.broadcast_to(b_ref[...], out_ref.shape)

    out_ref[...] += lax.dot_general(
        hs_ref[...], w_ref[...],
        dimension_numbers=(((1,), (1,)), ((), ())),
        preferred_element_type=jnp.float32,
    )


_NW = 32
_RCHUNK = 2


def _sortable(bits):
    m = lax.shift_right_logical(lax.shift_right_arithmetic(bits, 31), 1)
    return bits ^ m


def _sc_topk(h, B, H):
    rpw = B // _NW
    nv = H // 16
    mesh = plsc.VectorSubcoreMesh(
        core_axis_name="c", subcore_axis_name="s", num_cores=2, num_subcores=16)

    @functools.partial(
        pl.kernel,
        out_type=jax.ShapeDtypeStruct((B, H), jnp.float32),
        mesh=mesh,
        compiler_params=pltpu.CompilerParams(needs_layout_passes=False),
        scratch_types=[
            pltpu.VMEM((_RCHUNK, H), jnp.float32),
            pltpu.VMEM((_RCHUNK, H), jnp.float32),
            pltpu.VMEM((4096,), jnp.int32),
            pltpu.VMEM((_RCHUNK, H + 16), jnp.int32),
            pltpu.VMEM((_RCHUNK, H + 16), jnp.int32),
        ],
    )
    def sc_topk(h_hbm, out_hbm, inb, outb, hist, ck, ci):
        iota = lax.iota(jnp.int32, 16)
        iota_h = iota * 256
        zeros_i = jnp.zeros(16, jnp.int32)
        ones_i = jnp.ones(16, jnp.int32)
        zeros_f = jnp.zeros(16, jnp.float32)
        big = jnp.int32(2147483647)
        wid = lax.axis_index("s") * 2 + lax.axis_index("c")
        row0 = wid * rpw

        def zout(i, _):
            for r in range(_RCHUNK):
                outb[r, pl.ds(i * 16, 16)] = zeros_f
            return 0
        lax.fori_loop(0, nv, zout, 0)

        def zh(i, _):
            hist[pl.ds(i * 16, 16)] = zeros_i
            return 0
        lax.fori_loop(0, 256, zh, 0)

        def process(r):
            def p1(i, _):
                b = lax.bitcast_convert_type(inb[r, pl.ds(i * 16, 16)], jnp.int32)
                s = _sortable(b)
                bk = lax.shift_right_arithmetic(s, 24) + 128
                plsc.addupdate_scatter(hist, [iota_h + bk], ones_i)
                return 0
            lax.fori_loop(0, nv, p1, 0)

            carry = jnp.int32(0)
            bstar = jnp.int32(0)
            found = jnp.bool_(False)
            for v in range(15, -1, -1):
                acc = zeros_i
                for lane in range(16):
                    sl = pl.ds(lane * 256 + v * 16, 16)
                    acc = acc + hist[sl]
                    hist[sl] = zeros_i
                rc = lax.rev(plsc.cumsum(lax.rev(acc, (0,))), (0,)) + carry
                msk = rc >= _K
                pm = jnp.max(jnp.where(msk, iota + 1, 0))
                new = jnp.logical_and(pm > 0, jnp.logical_not(found))
                bstar = jnp.where(new, v * 16 + pm - 1, bstar)
                found = jnp.logical_or(found, pm > 0)
                carry = jnp.max(rc)

            bstar_s = lax.shift_left(bstar - 128, 24)

            rvec = jnp.full((16,), r, jnp.int32)

            def p2(i, off):
                b = lax.bitcast_convert_type(inb[r, pl.ds(i * 16, 16)], jnp.int32)
                s = _sortable(b)
                sel = s >= bstar_s
                idx = lax.shift_left(off, 4) + iota
                plsc.store_scatter(ck, [rvec, idx], s, mask=sel)
                plsc.store_scatter(ci, [rvec, idx], iota + i * 16, mask=sel)
                return off + jnp.where(sel, 1, 0)
            off = lax.fori_loop(0, nv, p2, zeros_i)
            maxlen = jnp.max(off)

            def bit_iter(bi, t):
                cand = t + lax.shift_left(jnp.int32(1), 23 - bi)

                def inner(j, cnt):
                    kv = ck[r, pl.ds(j * 16, 16)]
                    valid = off > j
                    return cnt + jnp.where(
                        jnp.logical_and(valid, kv >= cand), 1, 0)
                cntv = lax.fori_loop(0, maxlen, inner, zeros_i)
                return jnp.where(jnp.sum(cntv) >= _K, cand, t)
            t = lax.fori_loop(0, 24, bit_iter, bstar_s)

            def wri(j, _):
                kv = ck[r, pl.ds(j * 16, 16)]
                cols = ci[r, pl.ds(j * 16, 16)]
                keep = jnp.logical_and(off > j, kv >= t)
                val = lax.bitcast_convert_type(_sortable(kv), jnp.float32)
                plsc.store_scatter(outb, [rvec, cols], val, mask=keep)
                return 0
            lax.fori_loop(0, maxlen, wri, 0)

            def unw(j, _):
                cols = ci[r, pl.ds(j * 16, 16)]
                kv = ck[r, pl.ds(j * 16, 16)]
                keep = jnp.logical_and(off > j, kv >= t)
                plsc.store_scatter(outb, [rvec, cols], zeros_f, mask=keep)
                return 0
            return maxlen, unw

        def outer(rr, _):
            row = row0 + rr * _RCHUNK
            pltpu.sync_copy(h_hbm.at[pl.ds(row, _RCHUNK)], inb)
            states = [process(r) for r in range(_RCHUNK)]
            pltpu.sync_copy(outb, out_hbm.at[pl.ds(row, _RCHUNK)])
            for maxlen, unw in states:
                lax.fori_loop(0, maxlen, unw, 0)
            return 0
        lax.fori_loop(0, rpw // _RCHUNK, outer, 0)

    return sc_topk(h)


@jax.jit
def kernel(x, W_enc, b_enc, W_dec, b_dec):
    B, D = x.shape
    H = W_enc.shape[0]

    bm = min(1024, B)
    bn = min(1024, H)
    h = pl.pallas_call(
        _enc_body,
        grid=(B // bm, H // bn),
        in_specs=[
            pl.BlockSpec((bm, D), lambda i, j: (i, 0)),
            pl.BlockSpec((bn, D), lambda i, j: (j, 0)),
            pl.BlockSpec((1, bn), lambda i, j: (0, j)),
        ],
        out_specs=pl.BlockSpec((bm, bn), lambda i, j: (i, j)),
        out_shape=jax.ShapeDtypeStruct((B, H), jnp.float32),
    )(x, W_enc, b_enc.reshape(1, H))

    h_sparse = _sc_topk(h, B, H)

    bk = min(1024, H)
    recon = pl.pallas_call(
        _dec_body,
        grid=(B // bm, H // bk),
        in_specs=[
            pl.BlockSpec((bm, bk), lambda i, k: (i, k)),
            pl.BlockSpec((D, bk), lambda i, k: (0, k)),
            pl.BlockSpec((1, D), lambda i, k: (0, 0)),
        ],
        out_specs=pl.BlockSpec((bm, D), lambda i, k: (i, 0)),
        out_shape=jax.ShapeDtypeStruct((B, D), jnp.float32),
    )(h_sparse, W_dec, b_dec.reshape(1, D))

    return (h_sparse, recon)


# speedup vs baseline: 1.0711x; 1.0711x over previous
import functools

import jax
import jax.numpy as jnp
from jax import lax
from jax.experimental import pallas as pl
from jax.experimental.pallas import tpu as pltpu
from jax.experimental.pallas import tpu_sc as plsc

_K = 32


def _enc_body(x_ref, w_ref, b_ref, out_ref):
    acc = lax.dot_general(
        x_ref[...], w_ref[...],
        dimension_numbers=(((1,), (1,)), ((), ())),
        preferred_element_type=jnp.float32,
    )
    out_ref[...] = acc + b_ref[...]


def _dec_body(hs_ref, w_ref, b_ref, out_ref):
    k = pl.program_id(1)

    @pl.when(k == 0)
    def _():
        out_ref[...] = jnp.broadcast_to(b_ref[...], out_ref.shape)

    out_ref[...] += lax.dot_general(
        hs_ref[...], w_ref[...],
        dimension_numbers=(((1,), (1,)), ((), ())),
        preferred_element_type=jnp.float32,
    )


_NW = 32
_RCHUNK = 2


def _sortable(bits):
    m = lax.shift_right_logical(lax.shift_right_arithmetic(bits, 31), 1)
    return bits ^ m


def _sc_topk(h, B, H):
    rpw = B // _NW
    nv = H // 16
    mesh = plsc.VectorSubcoreMesh(
        core_axis_name="c", subcore_axis_name="s", num_cores=2, num_subcores=16)

    @functools.partial(
        pl.kernel,
        out_type=jax.ShapeDtypeStruct((B, H), jnp.float32),
        mesh=mesh,
        compiler_params=pltpu.CompilerParams(needs_layout_passes=False),
        scratch_types=[
            pltpu.VMEM((_RCHUNK, H), jnp.float32),
            pltpu.VMEM((_RCHUNK, H), jnp.float32),
            pltpu.VMEM((4096,), jnp.int32),
            pltpu.VMEM((_RCHUNK, H + 64), jnp.int32),
            pltpu.VMEM((_RCHUNK, H + 64), jnp.int32),
        ],
    )
    def sc_topk(h_hbm, out_hbm, inb, outb, hist, ck, ci):
        iota = lax.iota(jnp.int32, 16)
        iota_h = iota * 256
        zeros_i = jnp.zeros(16, jnp.int32)
        ones_i = jnp.ones(16, jnp.int32)
        zeros_f = jnp.zeros(16, jnp.float32)
        big = jnp.int32(2147483647)
        wid = lax.axis_index("s") * 2 + lax.axis_index("c")
        row0 = wid * rpw

        def zout(i, _):
            for u in range(8):
                for r in range(_RCHUNK):
                    outb[r, pl.ds((i * 8 + u) * 16, 16)] = zeros_f
            return 0
        lax.fori_loop(0, nv // 8, zout, 0)

        def zh(i, _):
            for u in range(8):
                hist[pl.ds((i * 8 + u) * 16, 16)] = zeros_i
            return 0
        lax.fori_loop(0, 32, zh, 0)

        def process(r):
            def p1(i, _):
                for u in range(8):
                    b = lax.bitcast_convert_type(
                        inb[r, pl.ds((i * 8 + u) * 16, 16)], jnp.int32)
                    s = _sortable(b)
                    bk = lax.shift_right_arithmetic(s, 24) + 128
                    plsc.addupdate_scatter(hist, [iota_h + bk], ones_i)
                return 0
            lax.fori_loop(0, nv // 8, p1, 0)

            carry = jnp.int32(0)
            bstar = jnp.int32(0)
            found = jnp.bool_(False)
            for v in range(15, -1, -1):
                acc = zeros_i
                for lane in range(16):
                    sl = pl.ds(lane * 256 + v * 16, 16)
                    acc = acc + hist[sl]
                    hist[sl] = zeros_i
                rc = lax.rev(plsc.cumsum(lax.rev(acc, (0,))), (0,)) + carry
                msk = rc >= _K
                pm = jnp.max(jnp.where(msk, iota + 1, 0))
                new = jnp.logical_and(pm > 0, jnp.logical_not(found))
                bstar = jnp.where(new, v * 16 + pm - 1, bstar)
                found = jnp.logical_or(found, pm > 0)
                carry = jnp.max(rc)

            bstar_s = lax.shift_left(bstar - 128, 24)

            rvec = jnp.full((16,), r, jnp.int32)

            def p2(i, off):
                for u in range(8):
                    b = lax.bitcast_convert_type(
                        inb[r, pl.ds((i * 8 + u) * 16, 16)], jnp.int32)
                    s = _sortable(b)
                    sel = s >= bstar_s
                    idx = lax.shift_left(off, 4) + iota
                    plsc.store_scatter(ck, [rvec, idx], s, mask=sel)
                    plsc.store_scatter(
                        ci, [rvec, idx], iota + (i * 8 + u) * 16, mask=sel)
                    off = off + jnp.where(sel, 1, 0)
                return off
            off = lax.fori_loop(0, nv // 8, p2, zeros_i)
            nj = (jnp.max(off) + 3) // 4

            def bit_iter(bi, t):
                cand = t + lax.shift_left(jnp.int32(1), 23 - bi)

                def inner(j4, cnt):
                    for u in range(4):
                        j = j4 * 4 + u
                        kv = ck[r, pl.ds(j * 16, 16)]
                        cnt = cnt + jnp.where(
                            jnp.logical_and(off > j, kv >= cand), 1, 0)
                    return cnt
                cntv = lax.fori_loop(0, nj, inner, zeros_i)
                return jnp.where(jnp.sum(cntv) >= _K, cand, t)
            t = lax.fori_loop(0, 24, bit_iter, bstar_s)

            def wri(j4, _):
                for u in range(4):
                    j = j4 * 4 + u
                    kv = ck[r, pl.ds(j * 16, 16)]
                    cols = ci[r, pl.ds(j * 16, 16)]
                    keep = jnp.logical_and(off > j, kv >= t)
                    val = lax.bitcast_convert_type(_sortable(kv), jnp.float32)
                    plsc.store_scatter(outb, [rvec, cols], val, mask=keep)
                return 0
            lax.fori_loop(0, nj, wri, 0)

            def unw(j4, _):
                for u in range(4):
                    j = j4 * 4 + u
                    kv = ck[r, pl.ds(j * 16, 16)]
                    cols = ci[r, pl.ds(j * 16, 16)]
                    keep = jnp.logical_and(off > j, kv >= t)
                    plsc.store_scatter(outb, [rvec, cols], zeros_f, mask=keep)
                return 0
            return nj, unw

        def outer(rr, _):
            row = row0 + rr * _RCHUNK
            pltpu.sync_copy(h_hbm.at[pl.ds(row, _RCHUNK)], inb)
            states = [process(r) for r in range(_RCHUNK)]
            pltpu.sync_copy(outb, out_hbm.at[pl.ds(row, _RCHUNK)])
            for maxlen, unw in states:
                lax.fori_loop(0, maxlen, unw, 0)
            return 0
        lax.fori_loop(0, rpw // _RCHUNK, outer, 0)

    return sc_topk(h)


@jax.jit
def kernel(x, W_enc, b_enc, W_dec, b_dec):
    B, D = x.shape
    H = W_enc.shape[0]

    bm = min(1024, B)
    bn = min(1024, H)
    h = pl.pallas_call(
        _enc_body,
        grid=(B // bm, H // bn),
        in_specs=[
            pl.BlockSpec((bm, D), lambda i, j: (i, 0)),
            pl.BlockSpec((bn, D), lambda i, j: (j, 0)),
            pl.BlockSpec((1, bn), lambda i, j: (0, j)),
        ],
        out_specs=pl.BlockSpec((bm, bn), lambda i, j: (i, j)),
        out_shape=jax.ShapeDtypeStruct((B, H), jnp.float32),
    )(x, W_enc, b_enc.reshape(1, H))

    h_sparse = _sc_topk(h, B, H)

    bk = min(1024, H)
    recon = pl.pallas_call(
        _dec_body,
        grid=(B // bm, H // bk),
        in_specs=[
            pl.BlockSpec((bm, bk), lambda i, k: (i, k)),
            pl.BlockSpec((D, bk), lambda i, k: (0, k)),
            pl.BlockSpec((1, D), lambda i, k: (0, 0)),
        ],
        out_specs=pl.BlockSpec((bm, D), lambda i, k: (i, 0)),
        out_shape=jax.ShapeDtypeStruct((B, D), jnp.float32),
    )(h_sparse, W_dec, b_dec.reshape(1, D))

    return (h_sparse, recon)


# speedup vs baseline: 2.5551x; 2.3856x over previous
import functools

import jax
import jax.numpy as jnp
from jax import lax
from jax.experimental import pallas as pl
from jax.experimental.pallas import tpu as pltpu
from jax.experimental.pallas import tpu_sc as plsc

_K = 32


def _enc_body(x_ref, w_ref, b_ref, out_ref):
    acc = lax.dot_general(
        x_ref[...], w_ref[...],
        dimension_numbers=(((1,), (1,)), ((), ())),
        preferred_element_type=jnp.float32,
    )
    out_ref[...] = acc + b_ref[...]


def _dec_body(hs_ref, w_ref, b_ref, out_ref):
    k = pl.program_id(1)

    @pl.when(k == 0)
    def _():
        out_ref[...] = jnp.broadcast_to(b_ref[...], out_ref.shape)

    out_ref[...] += lax.dot_general(
        hs_ref[...], w_ref[...],
        dimension_numbers=(((1,), (1,)), ((), ())),
        preferred_element_type=jnp.float32,
    )


def _tc_topk_body(h_ref, out_ref):
    h = h_ref[...]
    bm = h.shape[0]
    b = lax.bitcast_convert_type(h, jnp.int32)
    s = jnp.where(b < 0, b ^ jnp.int32(0x7FFFFFFF), b)

    def bit_step(i, t):
        cand = t + lax.shift_left(jnp.int32(1), 31 - i)
        cnt = jnp.sum((s >= cand[:, None]).astype(jnp.int32), axis=1)
        return jnp.where(cnt >= _K, cand, t)

    t0 = jnp.full((bm,), jnp.int32(-2147483647) - 1)
    t = lax.fori_loop(0, 32, bit_step, t0)
    out_ref[...] = jnp.where(s >= t[:, None], h, 0.0)


_NW = 32
_RCHUNK = 2


def _sortable(bits):
    m = lax.shift_right_logical(lax.shift_right_arithmetic(bits, 31), 1)
    return bits ^ m


def _sc_topk(h, B, H):
    rpw = B // _NW
    nv = H // 16
    mesh = plsc.VectorSubcoreMesh(
        core_axis_name="c", subcore_axis_name="s", num_cores=2, num_subcores=16)

    @functools.partial(
        pl.kernel,
        out_type=jax.ShapeDtypeStruct((B, H), jnp.float32),
        mesh=mesh,
        compiler_params=pltpu.CompilerParams(needs_layout_passes=False),
        scratch_types=[
            pltpu.VMEM((_RCHUNK, H), jnp.float32),
            pltpu.VMEM((_RCHUNK, H), jnp.float32),
            pltpu.VMEM((4096,), jnp.int32),
            pltpu.VMEM((_RCHUNK, H + 64), jnp.int32),
            pltpu.VMEM((_RCHUNK, H + 64), jnp.int32),
        ],
    )
    def sc_topk(h_hbm, out_hbm, inb, outb, hist, ck, ci):
        iota = lax.iota(jnp.int32, 16)
        iota_h = iota * 256
        zeros_i = jnp.zeros(16, jnp.int32)
        ones_i = jnp.ones(16, jnp.int32)
        zeros_f = jnp.zeros(16, jnp.float32)
        big = jnp.int32(2147483647)
        wid = lax.axis_index("s") * 2 + lax.axis_index("c")
        row0 = wid * rpw

        def zout(i, _):
            for u in range(8):
                for r in range(_RCHUNK):
                    outb[r, pl.ds((i * 8 + u) * 16, 16)] = zeros_f
            return 0
        lax.fori_loop(0, nv // 8, zout, 0)

        def zh(i, _):
            for u in range(8):
                hist[pl.ds((i * 8 + u) * 16, 16)] = zeros_i
            return 0
        lax.fori_loop(0, 32, zh, 0)

        def process(r):
            def p1(i, _):
                for u in range(8):
                    b = lax.bitcast_convert_type(
                        inb[r, pl.ds((i * 8 + u) * 16, 16)], jnp.int32)
                    s = _sortable(b)
                    bk = lax.shift_right_arithmetic(s, 24) + 128
                    plsc.addupdate_scatter(hist, [iota_h + bk], ones_i)
                return 0
            lax.fori_loop(0, nv // 8, p1, 0)

            carry = jnp.int32(0)
            bstar = jnp.int32(0)
            found = jnp.bool_(False)
            for v in range(15, -1, -1):
                acc = zeros_i
                for lane in range(16):
                    sl = pl.ds(lane * 256 + v * 16, 16)
                    acc = acc + hist[sl]
                    hist[sl] = zeros_i
                rc = lax.rev(plsc.cumsum(lax.rev(acc, (0,))), (0,)) + carry
                msk = rc >= _K
                pm = jnp.max(jnp.where(msk, iota + 1, 0))
                new = jnp.logical_and(pm > 0, jnp.logical_not(found))
                bstar = jnp.where(new, v * 16 + pm - 1, bstar)
                found = jnp.logical_or(found, pm > 0)
                carry = jnp.max(rc)

            bstar_s = lax.shift_left(bstar - 128, 24)

            rvec = jnp.full((16,), r, jnp.int32)

            def p2(i, off):
                for u in range(8):
                    b = lax.bitcast_convert_type(
                        inb[r, pl.ds((i * 8 + u) * 16, 16)], jnp.int32)
                    s = _sortable(b)
                    sel = s >= bstar_s
                    idx = lax.shift_left(off, 4) + iota
                    plsc.store_scatter(ck, [rvec, idx], s, mask=sel)
                    plsc.store_scatter(
                        ci, [rvec, idx], iota + (i * 8 + u) * 16, mask=sel)
                    off = off + jnp.where(sel, 1, 0)
                return off
            off = lax.fori_loop(0, nv // 8, p2, zeros_i)
            nj = (jnp.max(off) + 3) // 4

            def bit_iter(bi, t):
                cand = t + lax.shift_left(jnp.int32(1), 23 - bi)

                def inner(j4, cnt):
                    for u in range(4):
                        j = j4 * 4 + u
                        kv = ck[r, pl.ds(j * 16, 16)]
                        cnt = cnt + jnp.where(
                            jnp.logical_and(off > j, kv >= cand), 1, 0)
                    return cnt
                cntv = lax.fori_loop(0, nj, inner, zeros_i)
                return jnp.where(jnp.sum(cntv) >= _K, cand, t)
            t = lax.fori_loop(0, 24, bit_iter, bstar_s)

            def wri(j4, _):
                for u in range(4):
                    j = j4 * 4 + u
                    kv = ck[r, pl.ds(j * 16, 16)]
                    cols = ci[r, pl.ds(j * 16, 16)]
                    keep = jnp.logical_and(off > j, kv >= t)
                    val = lax.bitcast_convert_type(_sortable(kv), jnp.float32)
                    plsc.store_scatter(outb, [rvec, cols], val, mask=keep)
                return 0
            lax.fori_loop(0, nj, wri, 0)

            def unw(j4, _):
                for u in range(4):
                    j = j4 * 4 + u
                    kv = ck[r, pl.ds(j * 16, 16)]
                    cols = ci[r, pl.ds(j * 16, 16)]
                    keep = jnp.logical_and(off > j, kv >= t)
                    plsc.store_scatter(outb, [rvec, cols], zeros_f, mask=keep)
                return 0
            return nj, unw

        def outer(rr, _):
            row = row0 + rr * _RCHUNK
            pltpu.sync_copy(h_hbm.at[pl.ds(row, _RCHUNK)], inb)
            states = [process(r) for r in range(_RCHUNK)]
            pltpu.sync_copy(outb, out_hbm.at[pl.ds(row, _RCHUNK)])
            for maxlen, unw in states:
                lax.fori_loop(0, maxlen, unw, 0)
            return 0
        lax.fori_loop(0, rpw // _RCHUNK, outer, 0)

    return sc_topk(h)


@jax.jit
def kernel(x, W_enc, b_enc, W_dec, b_dec):
    B, D = x.shape
    H = W_enc.shape[0]

    bm = min(1024, B)
    bn = min(1024, H)
    h = pl.pallas_call(
        _enc_body,
        grid=(B // bm, H // bn),
        in_specs=[
            pl.BlockSpec((bm, D), lambda i, j: (i, 0)),
            pl.BlockSpec((bn, D), lambda i, j: (j, 0)),
            pl.BlockSpec((1, bn), lambda i, j: (0, j)),
        ],
        out_specs=pl.BlockSpec((bm, bn), lambda i, j: (i, j)),
        out_shape=jax.ShapeDtypeStruct((B, H), jnp.float32),
    )(x, W_enc, b_enc.reshape(1, H))

    b_sc = B // 4
    hs_sc = _sc_topk(h, b_sc, H)

    bt = 256
    hs_tc = pl.pallas_call(
        _tc_topk_body,
        grid=((B - b_sc) // bt,),
        in_specs=[pl.BlockSpec((bt, H), lambda i: (i + b_sc // bt, 0))],
        out_specs=pl.BlockSpec((bt, H), lambda i: (i, 0)),
        out_shape=jax.ShapeDtypeStruct((B - b_sc, H), jnp.float32),
    )(h)

    h_sparse = jnp.concatenate([hs_sc, hs_tc], axis=0)

    bk = min(1024, H)
    recon = pl.pallas_call(
        _dec_body,
        grid=(B // bm, H // bk),
        in_specs=[
            pl.BlockSpec((bm, bk), lambda i, k: (i, k)),
            pl.BlockSpec((D, bk), lambda i, k: (0, k)),
            pl.BlockSpec((1, D), lambda i, k: (0, 0)),
        ],
        out_specs=pl.BlockSpec((bm, D), lambda i, k: (i, 0)),
        out_shape=jax.ShapeDtypeStruct((B, D), jnp.float32),
    )(h_sparse, W_dec, b_dec.reshape(1, D))

    return (h_sparse, recon)


# speedup vs baseline: 2.5711x; 1.0063x over previous
import functools

import jax
import jax.numpy as jnp
from jax import lax
from jax.experimental import pallas as pl
from jax.experimental.pallas import tpu as pltpu
from jax.experimental.pallas import tpu_sc as plsc

_K = 32


def _enc_body(x_ref, w_ref, b_ref, out_ref):
    acc = lax.dot_general(
        x_ref[...], w_ref[...],
        dimension_numbers=(((1,), (1,)), ((), ())),
        preferred_element_type=jnp.float32,
    )
    out_ref[...] = acc + b_ref[...]


def _dec_body(hs_ref, w_ref, b_ref, out_ref):
    k = pl.program_id(1)

    @pl.when(k == 0)
    def _():
        out_ref[...] = jnp.broadcast_to(b_ref[...], out_ref.shape)

    out_ref[...] += lax.dot_general(
        hs_ref[...], w_ref[...],
        dimension_numbers=(((1,), (1,)), ((), ())),
        preferred_element_type=jnp.float32,
    )


def _tc_topk_body(h_ref, out_ref):
    h = h_ref[...]
    bm = h.shape[0]
    b = lax.bitcast_convert_type(h, jnp.int32)
    s = jnp.where(b < 0, b ^ jnp.int32(0x7FFFFFFF), b)

    def bit_step(i, t):
        cand = t + lax.shift_left(jnp.int32(1), 31 - i)
        cnt = jnp.sum((s >= cand[:, None]).astype(jnp.int32), axis=1)
        return jnp.where(cnt >= _K, cand, t)

    t0 = jnp.full((bm,), jnp.int32(-2147483647) - 1)
    t = lax.fori_loop(0, 32, bit_step, t0)
    out_ref[...] = jnp.where(s >= t[:, None], h, 0.0)


_NW = 32
_RCHUNK = 2


def _sortable(bits):
    m = lax.shift_right_logical(lax.shift_right_arithmetic(bits, 31), 1)
    return bits ^ m


def _sc_topk(h, B, H):
    rpw = B // _NW
    nv = H // 16
    mesh = plsc.VectorSubcoreMesh(
        core_axis_name="c", subcore_axis_name="s", num_cores=2, num_subcores=16)

    @functools.partial(
        pl.kernel,
        out_type=jax.ShapeDtypeStruct((B, H), jnp.float32),
        mesh=mesh,
        compiler_params=pltpu.CompilerParams(needs_layout_passes=False),
        scratch_types=[
            pltpu.VMEM((_RCHUNK, H), jnp.float32),
            pltpu.VMEM((_RCHUNK, H), jnp.float32),
            pltpu.VMEM((4096,), jnp.int32),
            pltpu.VMEM((_RCHUNK, H + 64), jnp.int32),
            pltpu.VMEM((_RCHUNK, H + 64), jnp.int32),
        ],
    )
    def sc_topk(h_hbm, out_hbm, inb, outb, hist, ck, ci):
        iota = lax.iota(jnp.int32, 16)
        iota_h = iota * 256
        zeros_i = jnp.zeros(16, jnp.int32)
        ones_i = jnp.ones(16, jnp.int32)
        zeros_f = jnp.zeros(16, jnp.float32)
        big = jnp.int32(2147483647)
        wid = lax.axis_index("s") * 2 + lax.axis_index("c")
        row0 = wid * rpw

        def zout(i, _):
            for u in range(8):
                for r in range(_RCHUNK):
                    outb[r, pl.ds((i * 8 + u) * 16, 16)] = zeros_f
            return 0
        lax.fori_loop(0, nv // 8, zout, 0)

        def zh(i, _):
            for u in range(8):
                hist[pl.ds((i * 8 + u) * 16, 16)] = zeros_i
            return 0
        lax.fori_loop(0, 32, zh, 0)

        def process(r):
            def p1(i, _):
                for u in range(8):
                    b = lax.bitcast_convert_type(
                        inb[r, pl.ds((i * 8 + u) * 16, 16)], jnp.int32)
                    s = _sortable(b)
                    bk = lax.shift_right_arithmetic(s, 24) + 128
                    plsc.addupdate_scatter(hist, [iota_h + bk], ones_i)
                return 0
            lax.fori_loop(0, nv // 8, p1, 0)

            carry = jnp.int32(0)
            bstar = jnp.int32(0)
            found = jnp.bool_(False)
            for v in range(15, -1, -1):
                acc = zeros_i
                for lane in range(16):
                    sl = pl.ds(lane * 256 + v * 16, 16)
                    acc = acc + hist[sl]
                    hist[sl] = zeros_i
                rc = lax.rev(plsc.cumsum(lax.rev(acc, (0,))), (0,)) + carry
                msk = rc >= _K
                pm = jnp.max(jnp.where(msk, iota + 1, 0))
                new = jnp.logical_and(pm > 0, jnp.logical_not(found))
                bstar = jnp.where(new, v * 16 + pm - 1, bstar)
                found = jnp.logical_or(found, pm > 0)
                carry = jnp.max(rc)

            bstar_s = lax.shift_left(bstar - 128, 24)

            rvec = jnp.full((16,), r, jnp.int32)

            def p2(i, off):
                for u in range(8):
                    b = lax.bitcast_convert_type(
                        inb[r, pl.ds((i * 8 + u) * 16, 16)], jnp.int32)
                    s = _sortable(b)
                    sel = s >= bstar_s
                    idx = lax.shift_left(off, 4) + iota
                    plsc.store_scatter(ck, [rvec, idx], s, mask=sel)
                    plsc.store_scatter(
                        ci, [rvec, idx], iota + (i * 8 + u) * 16, mask=sel)
                    off = off + jnp.where(sel, 1, 0)
                return off
            off = lax.fori_loop(0, nv // 8, p2, zeros_i)
            nj = (jnp.max(off) + 3) // 4

            def bit_iter(bi, t):
                cand = t + lax.shift_left(jnp.int32(1), 23 - bi)

                def inner(j4, cnt):
                    for u in range(4):
                        j = j4 * 4 + u
                        kv = ck[r, pl.ds(j * 16, 16)]
                        cnt = cnt + jnp.where(
                            jnp.logical_and(off > j, kv >= cand), 1, 0)
                    return cnt
                cntv = lax.fori_loop(0, nj, inner, zeros_i)
                return jnp.where(jnp.sum(cntv) >= _K, cand, t)
            t = lax.fori_loop(0, 24, bit_iter, bstar_s)

            def wri(j4, _):
                for u in range(4):
                    j = j4 * 4 + u
                    kv = ck[r, pl.ds(j * 16, 16)]
                    cols = ci[r, pl.ds(j * 16, 16)]
                    keep = jnp.logical_and(off > j, kv >= t)
                    val = lax.bitcast_convert_type(_sortable(kv), jnp.float32)
                    plsc.store_scatter(outb, [rvec, cols], val, mask=keep)
                return 0
            lax.fori_loop(0, nj, wri, 0)

            def unw(j4, _):
                for u in range(4):
                    j = j4 * 4 + u
                    kv = ck[r, pl.ds(j * 16, 16)]
                    cols = ci[r, pl.ds(j * 16, 16)]
                    keep = jnp.logical_and(off > j, kv >= t)
                    plsc.store_scatter(outb, [rvec, cols], zeros_f, mask=keep)
                return 0
            return nj, unw

        def outer(rr, _):
            row = row0 + rr * _RCHUNK
            pltpu.sync_copy(h_hbm.at[pl.ds(row, _RCHUNK)], inb)
            states = [process(r) for r in range(_RCHUNK)]
            pltpu.sync_copy(outb, out_hbm.at[pl.ds(row, _RCHUNK)])
            for maxlen, unw in states:
                lax.fori_loop(0, maxlen, unw, 0)
            return 0
        lax.fori_loop(0, rpw // _RCHUNK, outer, 0)

    return sc_topk(h)


@jax.jit
def kernel(x, W_enc, b_enc, W_dec, b_dec):
    B, D = x.shape
    H = W_enc.shape[0]

    bm = min(1024, B)
    bn = min(1024, H)
    h = pl.pallas_call(
        _enc_body,
        grid=(B // bm, H // bn),
        in_specs=[
            pl.BlockSpec((bm, D), lambda i, j: (i, 0)),
            pl.BlockSpec((bn, D), lambda i, j: (j, 0)),
            pl.BlockSpec((1, bn), lambda i, j: (0, j)),
        ],
        out_specs=pl.BlockSpec((bm, bn), lambda i, j: (i, j)),
        out_shape=jax.ShapeDtypeStruct((B, H), jnp.float32),
    )(x, W_enc, b_enc.reshape(1, H))

    b_sc = B // 4
    hs_sc = _sc_topk(h, b_sc, H)

    bt = 256
    hs_tc = pl.pallas_call(
        _tc_topk_body,
        grid=((B - b_sc) // bt,),
        in_specs=[pl.BlockSpec((bt, H), lambda i: (i + b_sc // bt, 0))],
        out_specs=pl.BlockSpec((bt, H), lambda i: (i, 0)),
        out_shape=jax.ShapeDtypeStruct((B - b_sc, H), jnp.float32),
    )(h)

    h_sparse = jnp.concatenate([hs_sc, hs_tc], axis=0)

    bk = min(1024, H)

    def dec(hs):
        rows = hs.shape[0]
        bmr = min(bm, rows)
        return pl.pallas_call(
            _dec_body,
            grid=(rows // bmr, H // bk),
            in_specs=[
                pl.BlockSpec((bmr, bk), lambda i, k: (i, k)),
                pl.BlockSpec((D, bk), lambda i, k: (0, k)),
                pl.BlockSpec((1, D), lambda i, k: (0, 0)),
            ],
            out_specs=pl.BlockSpec((bmr, D), lambda i, k: (i, 0)),
            out_shape=jax.ShapeDtypeStruct((rows, D), jnp.float32),
        )(hs, W_dec, b_dec.reshape(1, D))

    recon = jnp.concatenate([dec(hs_sc), dec(hs_tc)], axis=0)

    return (h_sparse, recon)
